# baseline (device time: 44357 ns/iter reference)
import jax
import jax.numpy as jnp
from jax import lax
from jax.experimental import pallas as pl
from jax.experimental.pallas import tpu as pltpu

N_DEV = 4
N_LAYERS = 3
N_STAGES = 2
N_CHUNKS = 4


def kernel(x, Win0, Wout0, Win1, Wout1, Win2, Wout2):
    b, d = x.shape
    dc = d // N_CHUNKS

    def body(
        x_ref,
        win0_ref,
        wout0_ref,
        win1_ref,
        wout1_ref,
        win2_ref,
        wout2_ref,
        out_ref,
        send_ref,
        recv_ref,
        send_sems,
        recv_sems,
    ):
        my_pos = lax.axis_index("i")
        partner = [my_pos ^ 1, 3 - my_pos]

        barrier_sem = pltpu.get_barrier_semaphore()
        for p in partner:
            pl.semaphore_signal(
                barrier_sem,
                inc=1,
                device_id=(p,),
                device_id_type=pl.DeviceIdType.MESH,
            )
        pl.semaphore_wait(barrier_sem, 2)

        wins = [win0_ref, win1_ref, win2_ref]
        wouts = [wout0_ref, wout1_ref, wout2_ref]

        prev = {}

        def exchange(layer, stage, chunk, value):
            if (stage, chunk) in prev:
                prev[stage, chunk].wait_send()
            send_ref[stage, chunk] = value
            s = (layer * N_STAGES + stage) * N_CHUNKS + chunk
            rdma = pltpu.make_async_remote_copy(
                src_ref=send_ref.at[stage, chunk],
                dst_ref=recv_ref.at[stage, chunk],
                send_sem=send_sems.at[s],
                recv_sem=recv_sems.at[s],
                device_id=(partner[stage],),
                device_id_type=pl.DeviceIdType.MESH,
            )
            rdma.start()
            prev[stage, chunk] = rdma
            return rdma

        h = jnp.maximum(
            jnp.dot(x_ref[:, :], win0_ref[:, :], preferred_element_type=jnp.float32),
            0.0,
        )
        for layer in range(N_LAYERS):
            wout = wouts[layer]

            p = []
            r1 = []
            for c in range(N_CHUNKS):
                pc = jnp.dot(
                    h, wout[:, c * dc : (c + 1) * dc],
                    preferred_element_type=jnp.float32,
                )
                p.append(pc)
                r1.append(exchange(layer, 0, c, pc))

            acc = []
            r2 = []
            for c in range(N_CHUNKS):
                r1[c].wait_recv()
                ac = p[c] + recv_ref[0, c]
                acc.append(ac)
                r2.append(exchange(layer, 1, c, ac))

            if layer == N_LAYERS - 1:
                for c in range(N_CHUNKS):
                    r2[c].wait_recv()
                    out_ref[:, c * dc : (c + 1) * dc] = acc[c] + recv_ref[1, c]
                for r in r1 + r2:
                    r.wait_send()
            else:
                win = wins[layer + 1]
                t = None
                for c in range(N_CHUNKS):
                    r2[c].wait_recv()
                    xc = acc[c] + recv_ref[1, c]
                    tc = jnp.dot(
                        xc, win[c * dc : (c + 1) * dc, :],
                        preferred_element_type=jnp.float32,
                    )
                    t = tc if t is None else t + tc
                h = jnp.maximum(t, 0.0)

    return pl.pallas_call(
        body,
        out_shape=jax.ShapeDtypeStruct((b, d), jnp.float32),
        in_specs=[pl.BlockSpec(memory_space=pltpu.VMEM)] * 7,
        out_specs=pl.BlockSpec(memory_space=pltpu.VMEM),
        scratch_shapes=[
            pltpu.VMEM((N_STAGES, N_CHUNKS, b, dc), jnp.float32),
            pltpu.VMEM((N_STAGES, N_CHUNKS, b, dc), jnp.float32),
            pltpu.SemaphoreType.DMA((N_LAYERS * N_STAGES * N_CHUNKS,)),
            pltpu.SemaphoreType.DMA((N_LAYERS * N_STAGES * N_CHUNKS,)),
        ],
        compiler_params=pltpu.CompilerParams(
            collective_id=0,
            vmem_limit_bytes=100 * 1024 * 1024,
        ),
    )(x, Win0, Wout0, Win1, Wout1, Win2, Wout2)
